# baseline (speedup 1.0000x reference)
import jax
import jax.numpy as jnp
from jax.experimental import pallas as pl
from jax.experimental.pallas import tpu as pltpu

B = 1024
T = 128
OBS = 256
A = 128

_BB = 64
_NSTEPS = B // _BB
_KO = 8


def _obs_bulk_copies(obs_in, obs_out, sem):
    return [
        pltpu.make_async_copy(obs_in.at[pl.ds(k * (B // _KO), B // _KO)],
                              obs_out.at[pl.ds(k * (B // _KO), B // _KO)],
                              sem.at[k])
        for k in range(_KO)
    ]


def _tc_body(step_smem,
             obs_in, obs_new,
             mask_in, act_in, rew_in, lp_in, val_in,
             mask_new, a_new, r_new, l_new, v_new,
             obs_out, mask_out, act_out, rew_out, lp_out, val_out,
             obs_fix, obs_nv, sem):
    s = step_smem[0]
    s8 = pl.multiple_of((s // 8) * 8, 8)
    pid = pl.program_id(0)

    @pl.when(pid == 0)
    def _start():
        for c in _obs_bulk_copies(obs_in, obs_out, sem):
            c.start()
        pltpu.make_async_copy(obs_in.at[:, pl.ds(s8, 8), :], obs_fix,
                              sem.at[_KO]).start()
        pltpu.make_async_copy(obs_new, obs_nv, sem.at[_KO + 1]).start()

    hit3 = jax.lax.broadcasted_iota(jnp.int32, (1, T, 1), 1) == s
    m_in = mask_in[...].astype(jnp.int8)
    m_new = mask_new[...].astype(jnp.int8)[:, None, :]
    mask_out[...] = jnp.where(hit3, m_new, m_in) != 0
    hit2 = jax.lax.broadcasted_iota(jnp.int32, (1, T), 1) == s
    act_out[...] = jnp.where(hit2, a_new[...], act_in[...])
    rew_out[...] = jnp.where(hit2, r_new[...], rew_in[...])
    lp_out[...] = jnp.where(hit2, l_new[...], lp_in[...])
    hit2v = jax.lax.broadcasted_iota(jnp.int32, (1, T + 1), 1) == s
    val_out[...] = jnp.where(hit2v, v_new[...], val_in[...])

    @pl.when(pid == _NSTEPS - 1)
    def _finish():
        pltpu.make_async_copy(obs_in.at[:, pl.ds(s8, 8), :], obs_fix,
                              sem.at[_KO]).wait()
        pltpu.make_async_copy(obs_new, obs_nv, sem.at[_KO + 1]).wait()
        r = s - s8
        hit8 = jax.lax.broadcasted_iota(jnp.int32, (1, 8, 1), 1) == r
        obs_fix[...] = jnp.where(hit8, obs_nv[...][:, None, :], obs_fix[...])
        for c in _obs_bulk_copies(obs_in, obs_out, sem):
            c.wait()
        st = pltpu.make_async_copy(obs_fix, obs_out.at[:, pl.ds(s8, 8), :],
                                   sem.at[_KO + 2])
        st.start()
        st.wait()


def kernel(state_obs, state_action_mask, state_actions, state_rewards,
           state_log_prob, state_values, state_advantages, state_targets,
           step, obs, action_mask, action, reward, log_prob, value):
    step_arr = jnp.asarray(step, jnp.int32).reshape((1,))

    any_spec = pl.BlockSpec(memory_space=pl.ANY)

    def b3(t_, a_):
        return pl.BlockSpec((_BB, t_, a_), lambda i: (i, 0, 0))

    def b2(t_):
        return pl.BlockSpec((_BB, t_), lambda i: (i, 0))

    outs = pl.pallas_call(
        _tc_body,
        grid=(_NSTEPS,),
        in_specs=[
            pl.BlockSpec(memory_space=pltpu.SMEM),
            any_spec, any_spec,
            b3(T, A), b2(T), b2(T), b2(T), b2(T + 1),
            b2(A), b2(1), b2(1), b2(1), b2(1),
        ],
        out_specs=[
            any_spec,
            b3(T, A), b2(T), b2(T), b2(T), b2(T + 1),
        ],
        out_shape=(
            jax.ShapeDtypeStruct((B, T, OBS), jnp.float32),
            jax.ShapeDtypeStruct((B, T, A), jnp.bool_),
            jax.ShapeDtypeStruct((B, T), jnp.int32),
            jax.ShapeDtypeStruct((B, T), jnp.float32),
            jax.ShapeDtypeStruct((B, T), jnp.float32),
            jax.ShapeDtypeStruct((B, T + 1), jnp.float32),
        ),
        scratch_shapes=[
            pltpu.VMEM((B, 8, OBS), jnp.float32),
            pltpu.VMEM((B, OBS), jnp.float32),
            pltpu.SemaphoreType.DMA((_KO + 3,)),
        ],
    )(step_arr, state_obs, obs,
      state_action_mask, state_actions, state_rewards, state_log_prob,
      state_values,
      action_mask,
      action.reshape(B, 1), reward.reshape(B, 1),
      log_prob.reshape(B, 1), value.reshape(B, 1))

    new_obs, new_mask, new_act, new_rew, new_lp, new_val = outs
    return (new_obs, new_mask, new_act, new_rew, new_lp, new_val,
            state_advantages, state_targets)


# speedup vs baseline: 21.4492x; 21.4492x over previous
import jax
import jax.numpy as jnp
from jax.experimental import pallas as pl
from jax.experimental.pallas import tpu as pltpu

B = 1024
T = 128
OBS = 256
A = 128

_BB = 64


def _big_body(step_ref, obs_in, mask_in, obs_new, mask_new,
              obs_out, mask_out):
    s = step_ref[0]
    hit3 = jax.lax.broadcasted_iota(jnp.int32, (1, T, 1), 1) == s
    obs_out[...] = jnp.where(hit3, obs_new[...][:, None, :], obs_in[...])
    m_in = mask_in[...].astype(jnp.int8)
    m_new = mask_new[...].astype(jnp.int8)[:, None, :]
    mask_out[...] = jnp.where(hit3, m_new, m_in) != 0


def _small_body(step_ref, act_in, rew_in, lp_in, val_in,
                a_new, r_new, l_new, v_new,
                act_out, rew_out, lp_out, val_out):
    s = step_ref[0]
    hit2 = jax.lax.broadcasted_iota(jnp.int32, (1, T), 1) == s
    act_out[...] = jnp.where(hit2, a_new[...], act_in[...])
    rew_out[...] = jnp.where(hit2, r_new[...], rew_in[...])
    lp_out[...] = jnp.where(hit2, l_new[...], lp_in[...])
    hit2v = jax.lax.broadcasted_iota(jnp.int32, (1, T + 1), 1) == s
    val_out[...] = jnp.where(hit2v, v_new[...], val_in[...])


def kernel(state_obs, state_action_mask, state_actions, state_rewards,
           state_log_prob, state_values, state_advantages, state_targets,
           step, obs, action_mask, action, reward, log_prob, value):
    step_arr = jnp.asarray(step, jnp.int32).reshape((1,))

    new_obs, new_mask = pl.pallas_call(
        _big_body,
        grid=(B // _BB,),
        in_specs=[
            pl.BlockSpec(memory_space=pltpu.SMEM),
            pl.BlockSpec((_BB, T, OBS), lambda i: (i, 0, 0)),
            pl.BlockSpec((_BB, T, A), lambda i: (i, 0, 0)),
            pl.BlockSpec((_BB, OBS), lambda i: (i, 0)),
            pl.BlockSpec((_BB, A), lambda i: (i, 0)),
        ],
        out_specs=[
            pl.BlockSpec((_BB, T, OBS), lambda i: (i, 0, 0)),
            pl.BlockSpec((_BB, T, A), lambda i: (i, 0, 0)),
        ],
        out_shape=(
            jax.ShapeDtypeStruct((B, T, OBS), jnp.float32),
            jax.ShapeDtypeStruct((B, T, A), jnp.bool_),
        ),
    )(step_arr, state_obs, state_action_mask, obs, action_mask)

    full2 = lambda t_: pl.BlockSpec((B, t_), lambda: (0, 0))
    new_act, new_rew, new_lp, new_val = pl.pallas_call(
        _small_body,
        in_specs=[
            pl.BlockSpec(memory_space=pltpu.SMEM),
            full2(T), full2(T), full2(T), full2(T + 1),
            full2(1), full2(1), full2(1), full2(1),
        ],
        out_specs=[full2(T), full2(T), full2(T), full2(T + 1)],
        out_shape=(
            jax.ShapeDtypeStruct((B, T), jnp.int32),
            jax.ShapeDtypeStruct((B, T), jnp.float32),
            jax.ShapeDtypeStruct((B, T), jnp.float32),
            jax.ShapeDtypeStruct((B, T + 1), jnp.float32),
        ),
    )(step_arr, state_actions, state_rewards, state_log_prob, state_values,
      action.reshape(B, 1), reward.reshape(B, 1),
      log_prob.reshape(B, 1), value.reshape(B, 1))

    return (new_obs, new_mask, new_act, new_rew, new_lp, new_val,
            state_advantages, state_targets)


# speedup vs baseline: 30.4924x; 1.4216x over previous
import jax
import jax.numpy as jnp
from jax.experimental import pallas as pl
from jax.experimental.pallas import tpu as pltpu

B = 1024
T = 128
OBS = 256
A = 128

_BB = 64


def _big_body(step_ref, obs_in, mask_in, obs_new, mask_new,
              obs_out, mask_out):
    s = step_ref[0]
    hit3 = jax.lax.broadcasted_iota(jnp.int32, (1, T, 1), 1) == s
    obs_out[...] = jnp.where(hit3, obs_new[...][:, None, :], obs_in[...])
    mask_out[...] = jnp.where(hit3, mask_new[...][:, None, :], mask_in[...])


def _small_body(step_ref, act_in, rew_in, lp_in, val_in,
                a_new, r_new, l_new, v_new,
                act_out, rew_out, lp_out, val_out):
    s = step_ref[0]
    hit2 = jax.lax.broadcasted_iota(jnp.int32, (1, T), 1) == s
    act_out[...] = jnp.where(hit2, a_new[...], act_in[...])
    rew_out[...] = jnp.where(hit2, r_new[...], rew_in[...])
    lp_out[...] = jnp.where(hit2, l_new[...], lp_in[...])
    hit2v = jax.lax.broadcasted_iota(jnp.int32, (1, T + 1), 1) == s
    val_out[...] = jnp.where(hit2v, v_new[...], val_in[...])


def kernel(state_obs, state_action_mask, state_actions, state_rewards,
           state_log_prob, state_values, state_advantages, state_targets,
           step, obs, action_mask, action, reward, log_prob, value):
    step_arr = jnp.asarray(step, jnp.int32).reshape((1,))

    new_obs, new_mask = pl.pallas_call(
        _big_body,
        grid=(B // _BB,),
        in_specs=[
            pl.BlockSpec(memory_space=pltpu.SMEM),
            pl.BlockSpec((_BB, T, OBS), lambda i: (i, 0, 0)),
            pl.BlockSpec((_BB, T, A), lambda i: (i, 0, 0)),
            pl.BlockSpec((_BB, OBS), lambda i: (i, 0)),
            pl.BlockSpec((_BB, A), lambda i: (i, 0)),
        ],
        out_specs=[
            pl.BlockSpec((_BB, T, OBS), lambda i: (i, 0, 0)),
            pl.BlockSpec((_BB, T, A), lambda i: (i, 0, 0)),
        ],
        out_shape=(
            jax.ShapeDtypeStruct((B, T, OBS), jnp.float32),
            jax.ShapeDtypeStruct((B, T, A), jnp.int8),
        ),
    )(step_arr, state_obs, state_action_mask.astype(jnp.int8),
      obs, action_mask.astype(jnp.int8))
    new_mask = new_mask.astype(jnp.bool_)

    full2 = lambda t_: pl.BlockSpec((B, t_), lambda: (0, 0))
    new_act, new_rew, new_lp, new_val = pl.pallas_call(
        _small_body,
        in_specs=[
            pl.BlockSpec(memory_space=pltpu.SMEM),
            full2(T), full2(T), full2(T), full2(T + 1),
            full2(1), full2(1), full2(1), full2(1),
        ],
        out_specs=[full2(T), full2(T), full2(T), full2(T + 1)],
        out_shape=(
            jax.ShapeDtypeStruct((B, T), jnp.int32),
            jax.ShapeDtypeStruct((B, T), jnp.float32),
            jax.ShapeDtypeStruct((B, T), jnp.float32),
            jax.ShapeDtypeStruct((B, T + 1), jnp.float32),
        ),
    )(step_arr, state_actions, state_rewards, state_log_prob, state_values,
      action.reshape(B, 1), reward.reshape(B, 1),
      log_prob.reshape(B, 1), value.reshape(B, 1))

    return (new_obs, new_mask, new_act, new_rew, new_lp, new_val,
            state_advantages, state_targets)
